# baseline (speedup 1.0000x reference)
import functools

import jax
import jax.numpy as jnp
from jax import lax
from jax.experimental import pallas as pl
from jax.experimental.pallas import tpu as pltpu
from jax.experimental.pallas import tpu_sc as plsc

HIDDEN = 64
LANES = 16
NCORES = 2
NSUBCORES = 16
NW = NCORES * NSUBCORES

SUB = 128
IRING = 4
ORING = 2

_GATHER_DNUMS = lax.GatherDimensionNumbers(
    offset_dims=(), collapsed_slice_dims=(0,), start_index_map=(0,)
)


def _perm(v, idx16):
    return lax.gather(
        v,
        idx16[:, None],
        _GATHER_DNUMS,
        slice_sizes=(1,),
        mode=lax.GatherScatterMode.PROMISE_IN_BOUNDS,
    )


def _rsqrt(s):
    i = lax.bitcast_convert_type(s, jnp.int32)
    i = jnp.int32(0x5F3759DF) - lax.shift_right_logical(i, 1)
    y = lax.bitcast_convert_type(i, jnp.float32)
    h = 0.5 * s
    for _ in range(1):
        y = y * (1.5 - h * y * y)
    return y


def _make_kernel(total_rows):
    per_w = total_rows // NW
    units = per_w // SUB
    groups = units // IRING
    mesh = plsc.VectorSubcoreMesh(core_axis_name="c", subcore_axis_name="s")

    @functools.partial(
        pl.kernel,
        mesh=mesh,
        out_type=jax.ShapeDtypeStruct((total_rows, HIDDEN), jnp.float32),
        scratch_types=[
            pltpu.VMEM((IRING, SUB), jnp.int32),
            pltpu.VMEM((IRING, SUB, HIDDEN), jnp.float32),
            pltpu.VMEM((ORING, SUB, HIDDEN), jnp.float32),
            [pltpu.SemaphoreType.DMA] * IRING,
            [pltpu.SemaphoreType.DMA] * IRING,
            [pltpu.SemaphoreType.DMA] * ORING,
        ],
        compiler_params=pltpu.CompilerParams(
            use_tc_tiling_on_sc=False, needs_layout_passes=False
        ),
    )
    def gather_norm(idx_hbm, table_hbm, out_hbm, idx_v, staged, obuf,
                    isems, gsems, ssems):
        wid = lax.axis_index("s") * NCORES + lax.axis_index("c")
        base = wid * per_w
        lane = lax.iota(jnp.int32, LANES)
        perms = [lane ^ (1 << k) for k in (3, 2, 1, 0)]

        def fire_idx(u, sl):
            pltpu.async_copy(
                idx_hbm.at[pl.ds(base + u * SUB, SUB)], idx_v.at[sl], isems[sl]
            )

        def wait_idx(sl):
            pltpu.make_async_copy(
                idx_hbm.at[pl.ds(base, SUB)], idx_v.at[sl], isems[sl]
            ).wait()

        def fire_gather(sl):
            pltpu.async_copy(
                table_hbm.at[idx_v.at[sl]], staged.at[sl], gsems[sl]
            )

        def wait_gather(sl):
            pltpu.make_async_copy(
                table_hbm.at[idx_v.at[sl]], staged.at[sl], gsems[sl]
            ).wait()

        def wait_store(so):
            pltpu.make_async_copy(
                obuf.at[so], out_hbm.at[pl.ds(base, SUB)], ssems[so]
            ).wait()

        for sl in range(IRING):
            fire_idx(sl, sl)
        for sl in range(IRING):
            wait_idx(sl)
            fire_gather(sl)

        def group_body(grp, carry):
            for b in range(IRING):
                u = grp * IRING + b
                so = b % ORING
                wait_gather(b)

                if b >= ORING:
                    wait_store(so)
                else:
                    @pl.when(grp > 0)
                    def _():
                        wait_store(so)

                @plsc.parallel_loop(0, SUB, step=1, unroll=16)
                def row_block(j):
                    v = [
                        staged[b, j, pl.ds(k * LANES, LANES)]
                        for k in range(4)
                    ]
                    q = v[0] * v[0] + v[1] * v[1] + v[2] * v[2] + v[3] * v[3]
                    for p in perms:
                        q = q + _perm(q, p)
                    y = _rsqrt(q)
                    for k in range(4):
                        obuf[so, j, pl.ds(k * LANES, LANES)] = v[k] * y

                pltpu.async_copy(
                    obuf.at[so],
                    out_hbm.at[pl.ds(base + u * SUB, SUB)],
                    ssems[so],
                )

                @pl.when(grp < groups - 1)
                def _():
                    fire_idx(u + IRING, b)
                    wait_idx(b)
                    fire_gather(b)
            return carry

        lax.fori_loop(0, groups, group_body, 0)

        for so in range(ORING):
            wait_store(so)

    return gather_norm


def kernel(inputs, table):
    total = inputs.size
    out = _make_kernel(total)(inputs.reshape(-1), table)
    return out.reshape(inputs.shape + (HIDDEN,))


# speedup vs baseline: 1.0584x; 1.0584x over previous
import functools

import jax
import jax.numpy as jnp
from jax import lax
from jax.experimental import pallas as pl
from jax.experimental.pallas import tpu as pltpu
from jax.experimental.pallas import tpu_sc as plsc

HIDDEN = 64
LANES = 16
NCORES = 2
NSUBCORES = 16
NW = NCORES * NSUBCORES

SUB = 128
IRING = 4
ORING = 2

_GATHER_DNUMS = lax.GatherDimensionNumbers(
    offset_dims=(), collapsed_slice_dims=(0,), start_index_map=(0,)
)


def _perm(v, idx16):
    return lax.gather(
        v,
        idx16[:, None],
        _GATHER_DNUMS,
        slice_sizes=(1,),
        mode=lax.GatherScatterMode.PROMISE_IN_BOUNDS,
    )


def _rsqrt(s):
    i = lax.bitcast_convert_type(s, jnp.int32)
    i = jnp.int32(0x5F3759DF) - lax.shift_right_logical(i, 1)
    y = lax.bitcast_convert_type(i, jnp.float32)
    h = 0.5 * s
    for _ in range(1):
        y = y * (1.5 - h * y * y)
    return y


def _make_kernel(total_rows):
    per_w = total_rows // NW
    units = per_w // SUB
    groups = units // IRING
    mesh = plsc.VectorSubcoreMesh(core_axis_name="c", subcore_axis_name="s")

    @functools.partial(
        pl.kernel,
        mesh=mesh,
        out_type=jax.ShapeDtypeStruct((total_rows, HIDDEN), jnp.float32),
        scratch_types=[
            pltpu.VMEM((IRING, SUB), jnp.int32),
            pltpu.VMEM((IRING, SUB, HIDDEN), jnp.float32),
            pltpu.VMEM((ORING, SUB, HIDDEN), jnp.float32),
            [pltpu.SemaphoreType.DMA] * IRING,
            [pltpu.SemaphoreType.DMA] * IRING,
            [pltpu.SemaphoreType.DMA] * ORING,
        ],
        compiler_params=pltpu.CompilerParams(
            use_tc_tiling_on_sc=False, needs_layout_passes=False
        ),
    )
    def gather_norm(idx_hbm, table_hbm, out_hbm, idx_v, staged, obuf,
                    isems, gsems, ssems):
        wid = lax.axis_index("s") * NCORES + lax.axis_index("c")
        base = wid * per_w
        lane = lax.iota(jnp.int32, LANES)
        perms = [lane ^ (1 << k) for k in (3, 2, 1, 0)]

        def fire_idx(u, sl):
            pltpu.async_copy(
                idx_hbm.at[pl.ds(base + u * SUB, SUB)], idx_v.at[sl], isems[sl]
            )

        def wait_idx(sl):
            pltpu.make_async_copy(
                idx_hbm.at[pl.ds(base, SUB)], idx_v.at[sl], isems[sl]
            ).wait()

        def fire_gather(sl):
            pltpu.async_copy(
                table_hbm.at[idx_v.at[sl]], staged.at[sl], gsems[sl]
            )

        def wait_gather(sl):
            pltpu.make_async_copy(
                table_hbm.at[idx_v.at[sl]], staged.at[sl], gsems[sl]
            ).wait()

        def wait_store(so):
            pltpu.make_async_copy(
                obuf.at[so], out_hbm.at[pl.ds(base, SUB)], ssems[so]
            ).wait()

        for sl in range(IRING):
            fire_idx(sl, sl)
        for sl in range(IRING):
            wait_idx(sl)
            fire_gather(sl)

        def group_body(grp, carry):
            for b in range(IRING):
                u = grp * IRING + b
                so = b % ORING
                wait_gather(b)

                if b >= ORING:
                    wait_store(so)
                else:
                    @pl.when(grp > 0)
                    def _():
                        wait_store(so)

                @plsc.parallel_loop(0, SUB, step=1, unroll=8)
                def row_block(j):
                    v = [
                        staged[b, j, pl.ds(k * LANES, LANES)]
                        for k in range(4)
                    ]
                    q = v[0] * v[0] + v[1] * v[1] + v[2] * v[2] + v[3] * v[3]
                    for p in perms:
                        q = q + _perm(q, p)
                    y = _rsqrt(q)
                    for k in range(4):
                        obuf[so, j, pl.ds(k * LANES, LANES)] = v[k] * y

                pltpu.async_copy(
                    obuf.at[so],
                    out_hbm.at[pl.ds(base + u * SUB, SUB)],
                    ssems[so],
                )

                @pl.when(grp < groups - 1)
                def _():
                    fire_idx(u + IRING, b)
                    wait_idx(b)
                    fire_gather(b)
            return carry

        lax.fori_loop(0, groups, group_body, 0)

        for so in range(ORING):
            wait_store(so)

    return gather_norm


def kernel(inputs, table):
    total = inputs.size
    out = _make_kernel(total)(inputs.reshape(-1), table)
    return out.reshape(inputs.shape + (HIDDEN,))


# speedup vs baseline: 1.0665x; 1.0077x over previous
import functools

import jax
import jax.numpy as jnp
from jax import lax
from jax.experimental import pallas as pl
from jax.experimental.pallas import tpu as pltpu
from jax.experimental.pallas import tpu_sc as plsc

HIDDEN = 64
LANES = 16
NCORES = 2
NSUBCORES = 16
NW = NCORES * NSUBCORES

SUB = 128
IRING = 4
ORING = 2

_GATHER_DNUMS = lax.GatherDimensionNumbers(
    offset_dims=(), collapsed_slice_dims=(0,), start_index_map=(0,)
)


def _perm(v, idx16):
    return lax.gather(
        v,
        idx16[:, None],
        _GATHER_DNUMS,
        slice_sizes=(1,),
        mode=lax.GatherScatterMode.PROMISE_IN_BOUNDS,
    )


def _rsqrt(s):
    i = lax.bitcast_convert_type(s, jnp.int32)
    i = jnp.int32(0x5F3759DF) - lax.shift_right_logical(i, 1)
    y = lax.bitcast_convert_type(i, jnp.float32)
    h = 0.5 * s
    for _ in range(1):
        y = y * (1.5 - h * y * y)
    return y


def _make_kernel(total_rows):
    per_w = total_rows // NW
    units = per_w // SUB
    groups = units // IRING
    mesh = plsc.VectorSubcoreMesh(core_axis_name="c", subcore_axis_name="s")

    @functools.partial(
        pl.kernel,
        mesh=mesh,
        out_type=jax.ShapeDtypeStruct((total_rows, HIDDEN), jnp.float32),
        scratch_types=[
            pltpu.VMEM((IRING, SUB), jnp.int32),
            pltpu.VMEM((IRING, SUB, HIDDEN), jnp.float32),
            pltpu.VMEM((ORING, SUB, HIDDEN), jnp.float32),
            [pltpu.SemaphoreType.DMA] * IRING,
            [pltpu.SemaphoreType.DMA] * IRING,
            [pltpu.SemaphoreType.DMA] * ORING,
        ],
        compiler_params=pltpu.CompilerParams(
            use_tc_tiling_on_sc=False, needs_layout_passes=False
        ),
    )
    def gather_norm(idx_hbm, table_hbm, out_hbm, idx_v, staged, obuf,
                    isems, gsems, ssems):
        wid = lax.axis_index("s") * NCORES + lax.axis_index("c")
        base = wid * per_w
        lane = lax.iota(jnp.int32, LANES)
        perms = [lane ^ (1 << k) for k in (3, 2, 1, 0)]

        def fire_idx(u, sl):
            pltpu.async_copy(
                idx_hbm.at[pl.ds(base + u * SUB, SUB)], idx_v.at[sl], isems[sl]
            )

        def wait_idx(sl):
            pltpu.make_async_copy(
                idx_hbm.at[pl.ds(base, SUB)], idx_v.at[sl], isems[sl]
            ).wait()

        def fire_gather(sl):
            pltpu.async_copy(
                table_hbm.at[idx_v.at[sl]], staged.at[sl], gsems[sl]
            )

        def wait_gather(sl):
            pltpu.make_async_copy(
                table_hbm.at[idx_v.at[sl]], staged.at[sl], gsems[sl]
            ).wait()

        def wait_store(so):
            pltpu.make_async_copy(
                obuf.at[so], out_hbm.at[pl.ds(base, SUB)], ssems[so]
            ).wait()

        for sl in range(IRING):
            fire_idx(sl, sl)
        for sl in range(IRING):
            wait_idx(sl)
            fire_gather(sl)

        def group_body(grp, carry):
            for b in range(IRING):
                u = grp * IRING + b
                so = b % ORING
                wait_gather(b)

                if b >= ORING:
                    wait_store(so)
                else:
                    @pl.when(grp > 0)
                    def _():
                        wait_store(so)

                @plsc.parallel_loop(0, SUB, step=1, unroll=8)
                def row_block(j):
                    v = [
                        staged[b, j, pl.ds(k * LANES, LANES)]
                        for k in range(4)
                    ]
                    q = v[0] * v[0] + v[1] * v[1] + v[2] * v[2] + v[3] * v[3]
                    s = jnp.sum(q)
                    y = _rsqrt(jnp.full((LANES,), s, jnp.float32))
                    for k in range(4):
                        obuf[so, j, pl.ds(k * LANES, LANES)] = v[k] * y

                pltpu.async_copy(
                    obuf.at[so],
                    out_hbm.at[pl.ds(base + u * SUB, SUB)],
                    ssems[so],
                )

                @pl.when(grp < groups - 1)
                def _():
                    fire_idx(u + IRING, b)
                    wait_idx(b)
                    fire_gather(b)
            return carry

        lax.fori_loop(0, groups, group_body, 0)

        for so in range(ORING):
            wait_store(so)

    return gather_norm


def kernel(inputs, table):
    total = inputs.size
    out = _make_kernel(total)(inputs.reshape(-1), table)
    return out.reshape(inputs.shape + (HIDDEN,))
